# baseline (speedup 1.0000x reference)
import functools

import jax
import jax.numpy as jnp
from jax import lax
from jax.experimental import pallas as pl
from jax.experimental.pallas import tpu as pltpu
from jax.experimental.pallas import tpu_sc as plsc

N = 10000
E = 320000
D = 128
NC = 2
NS = 16
NW = NC * NS
EPW = E // NW
C = 128
NFULL = EPW // C
PAIRS = NFULL // 2
REM = EPW - NFULL * C
RPT = N // 10
ZR = 40
DZ = N // 5


def _prep_body(h_ref, w_ref, ar_ref, wh_ref, s_ref):
    wh = jnp.dot(h_ref[...], w_ref[...], preferred_element_type=jnp.float32)
    wh_ref[...] = wh
    s_ref[...] = jnp.dot(wh, ar_ref[...], preferred_element_type=jnp.float32)


def _prep(h, W, ar):
    blk = 2000
    grid = N // blk
    return pl.pallas_call(
        _prep_body,
        grid=(grid,),
        in_specs=[
            pl.BlockSpec((blk, D), lambda i: (i, 0)),
            pl.BlockSpec((D, D), lambda i: (0, 0)),
            pl.BlockSpec((D, 2), lambda i: (0, 0)),
        ],
        out_specs=[
            pl.BlockSpec((blk, D), lambda i: (i, 0)),
            pl.BlockSpec((blk, 2), lambda i: (i, 0)),
        ],
        out_shape=[
            jax.ShapeDtypeStruct((N, D), jnp.float32),
            jax.ShapeDtypeStruct((N, 2), jnp.float32),
        ],
    )(h, W, ar)


def _edge_body(wh_hbm, s1_hbm, s2_hbm, row_hbm, col_hbm, part_hbm, dpart_hbm,
               idxr_a, idxc_a, s1g_a, s2g_a, ee_a, wrows_a,
               idxr_b, idxc_b, s1g_b, s2g_b, ee_b, wrows_b,
               idxr_r, idxc_r, s1g_r, s2g_r, ee_r, wrows_r,
               sidxr_a, sidxr_b, zrows, zd, acc, dacc,
               semg_a, semg_b, semi_a, semi_b, sems_a, sems_b, sems_r):
    cid = lax.axis_index("c")
    sid = lax.axis_index("s")
    wid = sid * NC + cid

    def _zero_accumulators():
        def _zrow(i, _):
            for j in range(D // 16):
                zrows[i, pl.ds(j * 16, 16)] = jnp.zeros((16,), jnp.float32)
            return 0
        lax.fori_loop(0, ZR, _zrow, 0)

        def _zd(i, _):
            zd[pl.ds(i * 16, 16)] = jnp.zeros((16,), jnp.float32)
            return 0
        lax.fori_loop(0, DZ // 16, _zd, 0)

        @pl.when(sid < 10)
        def _():
            for r in range(RPT // ZR):
                pltpu.sync_copy(zrows, acc.at[pl.ds(sid * RPT + r * ZR, ZR)])

        @pl.when(sid < 5)
        def _():
            pltpu.sync_copy(zd, dacc.at[pl.ds(sid * DZ, DZ)])

    base = wid * EPW

    def _scale_span(ee, wrows, g0, g1):
        @plsc.parallel_loop(g0, g1)
        def _scale(g):
            ev = ee[pl.ds(g * 16, 16)]
            for e in range(16):
                v = jnp.full((16,), ev[e], jnp.float32)
                eidx = g * 16 + e
                for j in range(D // 16):
                    wrows[eidx, pl.ds(j * 16, 16)] = (
                        wrows[eidx, pl.ds(j * 16, 16)] * v)

    def _compute_scatter(n, idxr, s1g, s2g, ee, wrows, sems):
        for j in range(n // 16):
            t = s1g[pl.ds(j * 16, 16)] + s2g[pl.ds(j * 16, 16)]
            ee[pl.ds(j * 16, 16)] = jnp.exp(jnp.maximum(t, 0.2 * t))
        pltpu.async_copy(ee, dacc.at[idxr], sems, add=True)
        _scale_span(ee, wrows, 0, n // 16)
        pltpu.async_copy(wrows, acc.at[idxr], sems, add=True)

    H = C // 2

    def _compute_scatter2(idxr2, s1g, s2g, ee, wrows, sems):
        for j in range(C // 16):
            t = s1g[pl.ds(j * 16, 16)] + s2g[pl.ds(j * 16, 16)]
            ee[pl.ds(j * 16, 16)] = jnp.exp(jnp.maximum(t, 0.2 * t))
        pltpu.async_copy(ee.at[pl.ds(0, H)], dacc.at[idxr2.at[0]], sems,
                         add=True)
        pltpu.async_copy(ee.at[pl.ds(H, H)], dacc.at[idxr2.at[1]], sems,
                         add=True)
        _scale_span(ee, wrows, 0, H // 16)
        pltpu.async_copy(wrows.at[pl.ds(0, H)], acc.at[idxr2.at[0]], sems,
                         add=True)
        _scale_span(ee, wrows, H // 16, C // 16)
        pltpu.async_copy(wrows.at[pl.ds(H, H)], acc.at[idxr2.at[1]], sems,
                         add=True)

    def _fire_idx(k, idxr, idxc, semi):
        off = base + k * C
        pltpu.async_copy(row_hbm.at[pl.ds(off, C)], idxr, semi)
        pltpu.async_copy(col_hbm.at[pl.ds(off, C)], idxc, semi)

    def _wait_idx(idxr, idxc, semi):
        pltpu.make_async_copy(row_hbm.at[pl.ds(0, C)], idxr, semi).wait()
        pltpu.make_async_copy(col_hbm.at[pl.ds(0, C)], idxc, semi).wait()

    def _fire_gathers(idxr, idxc, s1g, s2g, wrows, semg):
        pltpu.async_copy(s1_hbm.at[idxr], s1g, semg)
        pltpu.async_copy(s2_hbm.at[idxc], s2g, semg)
        pltpu.async_copy(wh_hbm.at[idxc], wrows, semg)

    def _wait_gathers(s1g, s2g, wrows, semg):
        pltpu.make_async_copy(s1_hbm.at[pl.ds(0, C)], s1g, semg).wait()
        pltpu.make_async_copy(s2_hbm.at[pl.ds(0, C)], s2g, semg).wait()
        pltpu.make_async_copy(wh_hbm.at[pl.ds(0, C)], wrows, semg).wait()

    def _wait_scatters(ee, wrows, sems, n):
        pltpu.make_async_copy(s1_hbm.at[pl.ds(0, n)], ee, sems).wait()
        pltpu.make_async_copy(wh_hbm.at[pl.ds(0, n)], wrows, sems).wait()

    offr = base + NFULL * C
    pltpu.sync_copy(row_hbm.at[pl.ds(offr, REM)], idxr_r)
    pltpu.sync_copy(col_hbm.at[pl.ds(offr, REM)], idxc_r)
    pltpu.sync_copy(s1_hbm.at[idxr_r], s1g_r)
    pltpu.sync_copy(s2_hbm.at[idxc_r], s2g_r)
    pltpu.sync_copy(wh_hbm.at[idxc_r], wrows_r)
    _fire_idx(0, idxr_a, idxc_a, semi_a)
    _wait_idx(idxr_a, idxc_a, semi_a)
    _fire_gathers(idxr_a, idxc_a, s1g_a, s2g_a, wrows_a, semg_a)
    _fire_idx(1, idxr_b, idxc_b, semi_b)

    _zero_accumulators()
    plsc.subcore_barrier()
    _compute_scatter(REM, idxr_r, s1g_r, s2g_r, ee_r, wrows_r, sems_r)

    def _copy_idx(idxr, sidxr):
        for j in range(H // 16):
            sidxr[0, pl.ds(j * 16, 16)] = idxr[pl.ds(j * 16, 16)]
            sidxr[1, pl.ds(j * 16, 16)] = idxr[pl.ds(H + j * 16, 16)]

    def _pair(i, _):
        ka = 2 * i
        kb = ka + 1
        _wait_gathers(s1g_a, s2g_a, wrows_a, semg_a)
        _copy_idx(idxr_a, sidxr_a)
        _wait_idx(idxr_b, idxc_b, semi_b)

        @pl.when(i > 0)
        def _():
            _wait_scatters(ee_b, wrows_b, sems_b, C)
        _fire_gathers(idxr_b, idxc_b, s1g_b, s2g_b, wrows_b, semg_b)

        @pl.when(i < PAIRS - 1)
        def _():
            _fire_idx(ka + 2, idxr_a, idxc_a, semi_a)
        _compute_scatter2(sidxr_a, s1g_a, s2g_a, ee_a, wrows_a, sems_a)
        _wait_gathers(s1g_b, s2g_b, wrows_b, semg_b)
        _copy_idx(idxr_b, sidxr_b)
        _wait_scatters(ee_a, wrows_a, sems_a, C)

        @pl.when(i < PAIRS - 1)
        def _():
            _wait_idx(idxr_a, idxc_a, semi_a)
            _fire_gathers(idxr_a, idxc_a, s1g_a, s2g_a, wrows_a, semg_a)
            _fire_idx(kb + 2, idxr_b, idxc_b, semi_b)
        _compute_scatter2(sidxr_b, s1g_b, s2g_b, ee_b, wrows_b, sems_b)
        return 0

    lax.fori_loop(0, PAIRS, _pair, 0)
    _wait_scatters(ee_b, wrows_b, sems_b, C)
    _wait_scatters(ee_r, wrows_r, sems_r, REM)
    plsc.subcore_barrier()

    @pl.when(sid < 10)
    def _():
        for r in range(RPT // ZR):
            off = sid * RPT + r * ZR
            pltpu.sync_copy(acc.at[pl.ds(off, ZR)], zrows)
            pltpu.sync_copy(zrows, part_hbm.at[cid, pl.ds(off, ZR)])

    @pl.when(sid < 5)
    def _():
        pltpu.sync_copy(dacc.at[pl.ds(sid * DZ, DZ)], zd)
        pltpu.sync_copy(zd, dpart_hbm.at[pl.ds(cid * N + sid * DZ, DZ)])


_edge_kernel = functools.partial(
    pl.kernel,
    out_type=[
        jax.ShapeDtypeStruct((NC, N, D), jnp.float32),
        jax.ShapeDtypeStruct((NC * N,), jnp.float32),
    ],
    mesh=plsc.VectorSubcoreMesh(core_axis_name="c", subcore_axis_name="s"),
    scratch_types=(
        [pltpu.VMEM((C,), jnp.int32),
         pltpu.VMEM((C,), jnp.int32),
         pltpu.VMEM((C,), jnp.float32),
         pltpu.VMEM((C,), jnp.float32),
         pltpu.VMEM((C,), jnp.float32),
         pltpu.VMEM((C, D), jnp.float32)]
        + [pltpu.VMEM((C,), jnp.int32),
           pltpu.VMEM((C,), jnp.int32),
           pltpu.VMEM((C,), jnp.float32),
           pltpu.VMEM((C,), jnp.float32),
           pltpu.VMEM((C,), jnp.float32),
           pltpu.VMEM((C, D), jnp.float32)]
        + [pltpu.VMEM((REM,), jnp.int32),
           pltpu.VMEM((REM,), jnp.int32),
           pltpu.VMEM((REM,), jnp.float32),
           pltpu.VMEM((REM,), jnp.float32),
           pltpu.VMEM((REM,), jnp.float32),
           pltpu.VMEM((REM, D), jnp.float32)]
        + [pltpu.VMEM((2, C // 2), jnp.int32),
           pltpu.VMEM((2, C // 2), jnp.int32)]
        + [pltpu.VMEM((ZR, D), jnp.float32),
           pltpu.VMEM((DZ,), jnp.float32),
           pltpu.VMEM_SHARED((N, D), jnp.float32),
           pltpu.VMEM_SHARED((N,), jnp.float32)]
        + [pltpu.SemaphoreType.DMA] * 7
    ),
)(_edge_body)


def _combine_body(p_ref, d_ref, o_ref):
    dsum = jnp.sum(d_ref[...], axis=1)
    r = 1.0 / (dsum + 1e-10)
    o_ref[...] = (p_ref[0] + p_ref[1]) * r[:, None]


def _combine(part, dT):
    blk = 2000
    grid = N // blk
    return pl.pallas_call(
        _combine_body,
        grid=(grid,),
        in_specs=[
            pl.BlockSpec((NC, blk, D), lambda i: (0, i, 0)),
            pl.BlockSpec((blk, NC), lambda i: (i, 0)),
        ],
        out_specs=pl.BlockSpec((blk, D), lambda i: (i, 0)),
        out_shape=jax.ShapeDtypeStruct((N, D), jnp.float32),
    )(part, dT)


def kernel(h, row, col, W, a):
    row32 = row.astype(jnp.int32)
    col32 = col.astype(jnp.int32)
    a1 = a[:D, 0]
    a2 = a[D:, 0]
    ar = jnp.stack([a1, a2], axis=1)
    wh, s = _prep(h, W, ar)
    s1 = s[:, 0]
    s2 = s[:, 1]
    part, dpart = _edge_kernel(wh, s1, s2, row32, col32)
    return _combine(part, dpart.reshape(NC, N).T)


# speedup vs baseline: 1.0427x; 1.0427x over previous
import functools

import jax
import jax.numpy as jnp
from jax import lax
from jax.experimental import pallas as pl
from jax.experimental.pallas import tpu as pltpu
from jax.experimental.pallas import tpu_sc as plsc

N = 10000
E = 320000
D = 128
NC = 2
NS = 16
NW = NC * NS
EPW = E // NW
C = 128
NFULL = EPW // C
PAIRS = NFULL // 2
REM = EPW - NFULL * C
RPT = N // 10
ZR = 40
DZ = N // 5


def _prep_body(h_ref, w_ref, ar_ref, wh_ref, s_ref):
    wh = jnp.dot(h_ref[...], w_ref[...], preferred_element_type=jnp.float32)
    wh_ref[...] = wh
    s_ref[...] = jnp.dot(wh, ar_ref[...], preferred_element_type=jnp.float32)


def _prep(h, W, ar):
    blk = 2000
    grid = N // blk
    return pl.pallas_call(
        _prep_body,
        grid=(grid,),
        in_specs=[
            pl.BlockSpec((blk, D), lambda i: (i, 0)),
            pl.BlockSpec((D, D), lambda i: (0, 0)),
            pl.BlockSpec((D, 2), lambda i: (0, 0)),
        ],
        out_specs=[
            pl.BlockSpec((blk, D), lambda i: (i, 0)),
            pl.BlockSpec((blk, 2), lambda i: (i, 0)),
        ],
        out_shape=[
            jax.ShapeDtypeStruct((N, D), jnp.float32),
            jax.ShapeDtypeStruct((N, 2), jnp.float32),
        ],
    )(h, W, ar)


def _edge_body(wh_hbm, s1_hbm, s2_hbm, row_hbm, col_hbm, part_hbm, dpart_hbm,
               idxr_a, idxc_a, s1g_a, s2g_a, ee_a, wrows_a,
               idxr_b, idxc_b, s1g_b, s2g_b, ee_b, wrows_b,
               idxr_r, idxc_r, s1g_r, s2g_r, ee_r, wrows_r,
               sidxr_a, sidxr_b, zrows, zd, acc, dacc,
               semg_a, semg_b, semi_a, semi_b, sems_a, sems_b, sems_r):
    cid = lax.axis_index("c")
    sid = lax.axis_index("s")
    wid = sid * NC + cid

    def _zero_accumulators():
        def _zrow(i, _):
            for j in range(D // 16):
                zrows[i, pl.ds(j * 16, 16)] = jnp.zeros((16,), jnp.float32)
            return 0
        lax.fori_loop(0, ZR, _zrow, 0)

        def _zd(i, _):
            zd[pl.ds(i * 16, 16)] = jnp.zeros((16,), jnp.float32)
            return 0
        lax.fori_loop(0, DZ // 16, _zd, 0)

        @pl.when(sid < 10)
        def _():
            for r in range(RPT // ZR):
                pltpu.sync_copy(zrows, acc.at[pl.ds(sid * RPT + r * ZR, ZR)])

        @pl.when(sid < 5)
        def _():
            pltpu.sync_copy(zd, dacc.at[pl.ds(sid * DZ, DZ)])

    base = wid * EPW

    def _scale_span(ee, wrows, g0, g1):
        def _scale(g, _):
            ev = ee[pl.ds(g * 16, 16)]
            for e in range(16):
                v = lax.gather(
                    ev, jnp.full((16, 1), e, jnp.int32),
                    lax.GatherDimensionNumbers(
                        offset_dims=(), collapsed_slice_dims=(0,),
                        start_index_map=(0,)),
                    (1,), mode=lax.GatherScatterMode.PROMISE_IN_BOUNDS)
                eidx = g * 16 + e
                for j in range(D // 16):
                    wrows[eidx, pl.ds(j * 16, 16)] = (
                        wrows[eidx, pl.ds(j * 16, 16)] * v)
            return 0
        lax.fori_loop(g0, g1, _scale, 0)

    def _compute_scatter(n, idxr, s1g, s2g, ee, wrows, sems):
        for j in range(n // 16):
            t = s1g[pl.ds(j * 16, 16)] + s2g[pl.ds(j * 16, 16)]
            ee[pl.ds(j * 16, 16)] = jnp.exp(jnp.maximum(t, 0.2 * t))
        pltpu.async_copy(ee, dacc.at[idxr], sems, add=True)
        _scale_span(ee, wrows, 0, n // 16)
        pltpu.async_copy(wrows, acc.at[idxr], sems, add=True)

    H = C // 2

    def _compute_scatter2(idxr2, s1g, s2g, ee, wrows, sems):
        for j in range(C // 16):
            t = s1g[pl.ds(j * 16, 16)] + s2g[pl.ds(j * 16, 16)]
            ee[pl.ds(j * 16, 16)] = jnp.exp(jnp.maximum(t, 0.2 * t))
        pltpu.async_copy(ee.at[pl.ds(0, H)], dacc.at[idxr2.at[0]], sems,
                         add=True)
        pltpu.async_copy(ee.at[pl.ds(H, H)], dacc.at[idxr2.at[1]], sems,
                         add=True)
        _scale_span(ee, wrows, 0, H // 16)
        pltpu.async_copy(wrows.at[pl.ds(0, H)], acc.at[idxr2.at[0]], sems,
                         add=True)
        _scale_span(ee, wrows, H // 16, C // 16)
        pltpu.async_copy(wrows.at[pl.ds(H, H)], acc.at[idxr2.at[1]], sems,
                         add=True)

    def _fire_idx(k, idxr, idxc, semi):
        off = base + k * C
        pltpu.async_copy(row_hbm.at[pl.ds(off, C)], idxr, semi)
        pltpu.async_copy(col_hbm.at[pl.ds(off, C)], idxc, semi)

    def _wait_idx(idxr, idxc, semi):
        pltpu.make_async_copy(row_hbm.at[pl.ds(0, C)], idxr, semi).wait()
        pltpu.make_async_copy(col_hbm.at[pl.ds(0, C)], idxc, semi).wait()

    def _fire_gathers(idxr, idxc, s1g, s2g, wrows, semg):
        pltpu.async_copy(s1_hbm.at[idxr], s1g, semg)
        pltpu.async_copy(s2_hbm.at[idxc], s2g, semg)
        pltpu.async_copy(wh_hbm.at[idxc], wrows, semg)

    def _wait_gathers(s1g, s2g, wrows, semg):
        pltpu.make_async_copy(s1_hbm.at[pl.ds(0, C)], s1g, semg).wait()
        pltpu.make_async_copy(s2_hbm.at[pl.ds(0, C)], s2g, semg).wait()
        pltpu.make_async_copy(wh_hbm.at[pl.ds(0, C)], wrows, semg).wait()

    def _wait_scatters(ee, wrows, sems, n):
        pltpu.make_async_copy(s1_hbm.at[pl.ds(0, n)], ee, sems).wait()
        pltpu.make_async_copy(wh_hbm.at[pl.ds(0, n)], wrows, sems).wait()

    offr = base + NFULL * C
    pltpu.sync_copy(row_hbm.at[pl.ds(offr, REM)], idxr_r)
    pltpu.sync_copy(col_hbm.at[pl.ds(offr, REM)], idxc_r)
    pltpu.sync_copy(s1_hbm.at[idxr_r], s1g_r)
    pltpu.sync_copy(s2_hbm.at[idxc_r], s2g_r)
    pltpu.sync_copy(wh_hbm.at[idxc_r], wrows_r)
    _fire_idx(0, idxr_a, idxc_a, semi_a)
    _wait_idx(idxr_a, idxc_a, semi_a)
    _fire_gathers(idxr_a, idxc_a, s1g_a, s2g_a, wrows_a, semg_a)
    _fire_idx(1, idxr_b, idxc_b, semi_b)

    _zero_accumulators()
    plsc.subcore_barrier()
    _compute_scatter(REM, idxr_r, s1g_r, s2g_r, ee_r, wrows_r, sems_r)

    def _copy_idx(idxr, sidxr):
        for j in range(H // 16):
            sidxr[0, pl.ds(j * 16, 16)] = idxr[pl.ds(j * 16, 16)]
            sidxr[1, pl.ds(j * 16, 16)] = idxr[pl.ds(H + j * 16, 16)]

    def _pair(i, _):
        ka = 2 * i
        kb = ka + 1
        _wait_gathers(s1g_a, s2g_a, wrows_a, semg_a)
        _copy_idx(idxr_a, sidxr_a)
        _wait_idx(idxr_b, idxc_b, semi_b)

        @pl.when(i > 0)
        def _():
            _wait_scatters(ee_b, wrows_b, sems_b, C)
        _fire_gathers(idxr_b, idxc_b, s1g_b, s2g_b, wrows_b, semg_b)

        @pl.when(i < PAIRS - 1)
        def _():
            _fire_idx(ka + 2, idxr_a, idxc_a, semi_a)
        _compute_scatter2(sidxr_a, s1g_a, s2g_a, ee_a, wrows_a, sems_a)
        _wait_gathers(s1g_b, s2g_b, wrows_b, semg_b)
        _copy_idx(idxr_b, sidxr_b)
        _wait_scatters(ee_a, wrows_a, sems_a, C)

        @pl.when(i < PAIRS - 1)
        def _():
            _wait_idx(idxr_a, idxc_a, semi_a)
            _fire_gathers(idxr_a, idxc_a, s1g_a, s2g_a, wrows_a, semg_a)
            _fire_idx(kb + 2, idxr_b, idxc_b, semi_b)
        _compute_scatter2(sidxr_b, s1g_b, s2g_b, ee_b, wrows_b, sems_b)
        return 0

    lax.fori_loop(0, PAIRS, _pair, 0)
    _wait_scatters(ee_b, wrows_b, sems_b, C)
    _wait_scatters(ee_r, wrows_r, sems_r, REM)
    plsc.subcore_barrier()

    @pl.when(sid < 10)
    def _():
        for r in range(RPT // ZR):
            off = sid * RPT + r * ZR
            pltpu.sync_copy(acc.at[pl.ds(off, ZR)], zrows)
            pltpu.sync_copy(zrows, part_hbm.at[cid, pl.ds(off, ZR)])

    @pl.when(sid < 5)
    def _():
        pltpu.sync_copy(dacc.at[pl.ds(sid * DZ, DZ)], zd)
        pltpu.sync_copy(zd, dpart_hbm.at[pl.ds(cid * N + sid * DZ, DZ)])


_edge_kernel = functools.partial(
    pl.kernel,
    out_type=[
        jax.ShapeDtypeStruct((NC, N, D), jnp.float32),
        jax.ShapeDtypeStruct((NC * N,), jnp.float32),
    ],
    mesh=plsc.VectorSubcoreMesh(core_axis_name="c", subcore_axis_name="s"),
    scratch_types=(
        [pltpu.VMEM((C,), jnp.int32),
         pltpu.VMEM((C,), jnp.int32),
         pltpu.VMEM((C,), jnp.float32),
         pltpu.VMEM((C,), jnp.float32),
         pltpu.VMEM((C,), jnp.float32),
         pltpu.VMEM((C, D), jnp.float32)]
        + [pltpu.VMEM((C,), jnp.int32),
           pltpu.VMEM((C,), jnp.int32),
           pltpu.VMEM((C,), jnp.float32),
           pltpu.VMEM((C,), jnp.float32),
           pltpu.VMEM((C,), jnp.float32),
           pltpu.VMEM((C, D), jnp.float32)]
        + [pltpu.VMEM((REM,), jnp.int32),
           pltpu.VMEM((REM,), jnp.int32),
           pltpu.VMEM((REM,), jnp.float32),
           pltpu.VMEM((REM,), jnp.float32),
           pltpu.VMEM((REM,), jnp.float32),
           pltpu.VMEM((REM, D), jnp.float32)]
        + [pltpu.VMEM((2, C // 2), jnp.int32),
           pltpu.VMEM((2, C // 2), jnp.int32)]
        + [pltpu.VMEM((ZR, D), jnp.float32),
           pltpu.VMEM((DZ,), jnp.float32),
           pltpu.VMEM_SHARED((N, D), jnp.float32),
           pltpu.VMEM_SHARED((N,), jnp.float32)]
        + [pltpu.SemaphoreType.DMA] * 7
    ),
)(_edge_body)


def _combine_body(p_ref, d_ref, o_ref):
    dsum = jnp.sum(d_ref[...], axis=1)
    r = 1.0 / (dsum + 1e-10)
    o_ref[...] = (p_ref[0] + p_ref[1]) * r[:, None]


def _combine(part, dT):
    blk = 2000
    grid = N // blk
    return pl.pallas_call(
        _combine_body,
        grid=(grid,),
        in_specs=[
            pl.BlockSpec((NC, blk, D), lambda i: (0, i, 0)),
            pl.BlockSpec((blk, NC), lambda i: (i, 0)),
        ],
        out_specs=pl.BlockSpec((blk, D), lambda i: (i, 0)),
        out_shape=jax.ShapeDtypeStruct((N, D), jnp.float32),
    )(part, dT)


def kernel(h, row, col, W, a):
    row32 = row.astype(jnp.int32)
    col32 = col.astype(jnp.int32)
    a1 = a[:D, 0]
    a2 = a[D:, 0]
    ar = jnp.stack([a1, a2], axis=1)
    wh, s = _prep(h, W, ar)
    s1 = s[:, 0]
    s2 = s[:, 1]
    part, dpart = _edge_kernel(wh, s1, s2, row32, col32)
    return _combine(part, dpart.reshape(NC, N).T)


# speedup vs baseline: 1.0458x; 1.0030x over previous
import functools

import jax
import jax.numpy as jnp
from jax import lax
from jax.experimental import pallas as pl
from jax.experimental.pallas import tpu as pltpu
from jax.experimental.pallas import tpu_sc as plsc

N = 10000
E = 320000
D = 128
NC = 2
NS = 16
NW = NC * NS
EPW = E // NW
C = 128
NFULL = EPW // C
PAIRS = NFULL // 2
REM = EPW - NFULL * C
RPT = N // 10
ZR = 40
DZ = N // 5


def _prep_body(h_ref, w_ref, a_ref, wh_ref, s_ref):
    wh = jnp.dot(h_ref[...], w_ref[...], preferred_element_type=jnp.float32)
    wh_ref[...] = wh
    ar = jnp.concatenate([a_ref[:D, :], a_ref[D:, :]], axis=1)
    s_ref[...] = jnp.dot(wh, ar, preferred_element_type=jnp.float32)


def _prep(h, W, a):
    blk = 2000
    grid = N // blk
    return pl.pallas_call(
        _prep_body,
        grid=(grid,),
        in_specs=[
            pl.BlockSpec((blk, D), lambda i: (i, 0)),
            pl.BlockSpec((D, D), lambda i: (0, 0)),
            pl.BlockSpec((2 * D, 1), lambda i: (0, 0)),
        ],
        out_specs=[
            pl.BlockSpec((blk, D), lambda i: (i, 0)),
            pl.BlockSpec((blk, 2), lambda i: (i, 0)),
        ],
        out_shape=[
            jax.ShapeDtypeStruct((N, D), jnp.float32),
            jax.ShapeDtypeStruct((N, 2), jnp.float32),
        ],
    )(h, W, a)


def _edge_body(wh_hbm, s1_hbm, s2_hbm, row_hbm, col_hbm, part_hbm, dpart_hbm,
               idxr_a, idxc_a, s1g_a, s2g_a, ee_a, wrows_a,
               idxr_b, idxc_b, s1g_b, s2g_b, ee_b, wrows_b,
               idxr_r, idxc_r, s1g_r, s2g_r, ee_r, wrows_r,
               sidxr_a, sidxr_b, zrows, zd, acc, dacc,
               semg_a, semg_b, semi_a, semi_b, sems_a, sems_b, sems_r):
    cid = lax.axis_index("c")
    sid = lax.axis_index("s")
    wid = sid * NC + cid

    def _zero_accumulators():
        def _zrow(i, _):
            for j in range(D // 16):
                zrows[i, pl.ds(j * 16, 16)] = jnp.zeros((16,), jnp.float32)
            return 0
        lax.fori_loop(0, ZR, _zrow, 0)

        def _zd(i, _):
            zd[pl.ds(i * 16, 16)] = jnp.zeros((16,), jnp.float32)
            return 0
        lax.fori_loop(0, DZ // 16, _zd, 0)

        @pl.when(sid < 10)
        def _():
            for r in range(RPT // ZR):
                pltpu.sync_copy(zrows, acc.at[pl.ds(sid * RPT + r * ZR, ZR)])

        @pl.when(sid < 5)
        def _():
            pltpu.sync_copy(zd, dacc.at[pl.ds(sid * DZ, DZ)])

    base = wid * EPW

    def _scale_span(ee, wrows, g0, g1):
        def _scale(g, _):
            ev = ee[pl.ds(g * 16, 16)]
            for e in range(16):
                v = lax.gather(
                    ev, jnp.full((16, 1), e, jnp.int32),
                    lax.GatherDimensionNumbers(
                        offset_dims=(), collapsed_slice_dims=(0,),
                        start_index_map=(0,)),
                    (1,), mode=lax.GatherScatterMode.PROMISE_IN_BOUNDS)
                eidx = g * 16 + e
                for j in range(D // 16):
                    wrows[eidx, pl.ds(j * 16, 16)] = (
                        wrows[eidx, pl.ds(j * 16, 16)] * v)
            return 0
        lax.fori_loop(g0, g1, _scale, 0)

    def _compute_scatter(n, idxr, s1g, s2g, ee, wrows, sems):
        for j in range(n // 16):
            t = s1g[pl.ds(j * 16, 16)] + s2g[pl.ds(j * 16, 16)]
            ee[pl.ds(j * 16, 16)] = jnp.exp(jnp.maximum(t, 0.2 * t))
        pltpu.async_copy(ee, dacc.at[idxr], sems, add=True)
        _scale_span(ee, wrows, 0, n // 16)
        pltpu.async_copy(wrows, acc.at[idxr], sems, add=True)

    H = C // 2

    def _compute_scatter2(idxr2, s1g, s2g, ee, wrows, sems):
        for j in range(C // 16):
            t = s1g[pl.ds(j * 16, 16)] + s2g[pl.ds(j * 16, 16)]
            ee[pl.ds(j * 16, 16)] = jnp.exp(jnp.maximum(t, 0.2 * t))
        pltpu.async_copy(ee.at[pl.ds(0, H)], dacc.at[idxr2.at[0]], sems,
                         add=True)
        pltpu.async_copy(ee.at[pl.ds(H, H)], dacc.at[idxr2.at[1]], sems,
                         add=True)
        _scale_span(ee, wrows, 0, H // 16)
        pltpu.async_copy(wrows.at[pl.ds(0, H)], acc.at[idxr2.at[0]], sems,
                         add=True)
        _scale_span(ee, wrows, H // 16, C // 16)
        pltpu.async_copy(wrows.at[pl.ds(H, H)], acc.at[idxr2.at[1]], sems,
                         add=True)

    def _fire_idx(k, idxr, idxc, semi):
        off = base + k * C
        pltpu.async_copy(row_hbm.at[pl.ds(off, C)], idxr, semi)
        pltpu.async_copy(col_hbm.at[pl.ds(off, C)], idxc, semi)

    def _wait_idx(idxr, idxc, semi):
        pltpu.make_async_copy(row_hbm.at[pl.ds(0, C)], idxr, semi).wait()
        pltpu.make_async_copy(col_hbm.at[pl.ds(0, C)], idxc, semi).wait()

    def _fire_gathers(idxr, idxc, s1g, s2g, wrows, semg):
        pltpu.async_copy(s1_hbm.at[idxr], s1g, semg)
        pltpu.async_copy(s2_hbm.at[idxc], s2g, semg)
        pltpu.async_copy(wh_hbm.at[idxc], wrows, semg)

    def _wait_gathers(s1g, s2g, wrows, semg):
        pltpu.make_async_copy(s1_hbm.at[pl.ds(0, C)], s1g, semg).wait()
        pltpu.make_async_copy(s2_hbm.at[pl.ds(0, C)], s2g, semg).wait()
        pltpu.make_async_copy(wh_hbm.at[pl.ds(0, C)], wrows, semg).wait()

    def _wait_scatters(ee, wrows, sems, n):
        pltpu.make_async_copy(s1_hbm.at[pl.ds(0, n)], ee, sems).wait()
        pltpu.make_async_copy(wh_hbm.at[pl.ds(0, n)], wrows, sems).wait()

    offr = base + NFULL * C
    pltpu.sync_copy(row_hbm.at[pl.ds(offr, REM)], idxr_r)
    pltpu.sync_copy(col_hbm.at[pl.ds(offr, REM)], idxc_r)
    pltpu.sync_copy(s1_hbm.at[idxr_r], s1g_r)
    pltpu.sync_copy(s2_hbm.at[idxc_r], s2g_r)
    pltpu.sync_copy(wh_hbm.at[idxc_r], wrows_r)
    _fire_idx(0, idxr_a, idxc_a, semi_a)
    _wait_idx(idxr_a, idxc_a, semi_a)
    _fire_gathers(idxr_a, idxc_a, s1g_a, s2g_a, wrows_a, semg_a)
    _fire_idx(1, idxr_b, idxc_b, semi_b)

    _zero_accumulators()
    plsc.subcore_barrier()
    _compute_scatter(REM, idxr_r, s1g_r, s2g_r, ee_r, wrows_r, sems_r)

    def _copy_idx(idxr, sidxr):
        for j in range(H // 16):
            sidxr[0, pl.ds(j * 16, 16)] = idxr[pl.ds(j * 16, 16)]
            sidxr[1, pl.ds(j * 16, 16)] = idxr[pl.ds(H + j * 16, 16)]

    def _pair(i, _):
        ka = 2 * i
        kb = ka + 1
        _wait_gathers(s1g_a, s2g_a, wrows_a, semg_a)
        _copy_idx(idxr_a, sidxr_a)
        _wait_idx(idxr_b, idxc_b, semi_b)

        @pl.when(i > 0)
        def _():
            _wait_scatters(ee_b, wrows_b, sems_b, C)
        _fire_gathers(idxr_b, idxc_b, s1g_b, s2g_b, wrows_b, semg_b)

        @pl.when(i < PAIRS - 1)
        def _():
            _fire_idx(ka + 2, idxr_a, idxc_a, semi_a)
        _compute_scatter2(sidxr_a, s1g_a, s2g_a, ee_a, wrows_a, sems_a)
        _wait_gathers(s1g_b, s2g_b, wrows_b, semg_b)
        _copy_idx(idxr_b, sidxr_b)
        _wait_scatters(ee_a, wrows_a, sems_a, C)

        @pl.when(i < PAIRS - 1)
        def _():
            _wait_idx(idxr_a, idxc_a, semi_a)
            _fire_gathers(idxr_a, idxc_a, s1g_a, s2g_a, wrows_a, semg_a)
            _fire_idx(kb + 2, idxr_b, idxc_b, semi_b)
        _compute_scatter2(sidxr_b, s1g_b, s2g_b, ee_b, wrows_b, sems_b)
        return 0

    lax.fori_loop(0, PAIRS, _pair, 0)
    _wait_scatters(ee_b, wrows_b, sems_b, C)
    _wait_scatters(ee_r, wrows_r, sems_r, REM)
    plsc.subcore_barrier()

    @pl.when(sid < 10)
    def _():
        for r in range(RPT // ZR):
            off = sid * RPT + r * ZR
            pltpu.sync_copy(acc.at[pl.ds(off, ZR)], zrows)
            pltpu.sync_copy(zrows, part_hbm.at[cid, pl.ds(off, ZR)])

    @pl.when(sid < 5)
    def _():
        pltpu.sync_copy(dacc.at[pl.ds(sid * DZ, DZ)], zd)
        pltpu.sync_copy(zd, dpart_hbm.at[pl.ds(cid * N + sid * DZ, DZ)])


_edge_kernel = functools.partial(
    pl.kernel,
    out_type=[
        jax.ShapeDtypeStruct((NC, N, D), jnp.float32),
        jax.ShapeDtypeStruct((NC * N,), jnp.float32),
    ],
    mesh=plsc.VectorSubcoreMesh(core_axis_name="c", subcore_axis_name="s"),
    scratch_types=(
        [pltpu.VMEM((C,), jnp.int32),
         pltpu.VMEM((C,), jnp.int32),
         pltpu.VMEM((C,), jnp.float32),
         pltpu.VMEM((C,), jnp.float32),
         pltpu.VMEM((C,), jnp.float32),
         pltpu.VMEM((C, D), jnp.float32)]
        + [pltpu.VMEM((C,), jnp.int32),
           pltpu.VMEM((C,), jnp.int32),
           pltpu.VMEM((C,), jnp.float32),
           pltpu.VMEM((C,), jnp.float32),
           pltpu.VMEM((C,), jnp.float32),
           pltpu.VMEM((C, D), jnp.float32)]
        + [pltpu.VMEM((REM,), jnp.int32),
           pltpu.VMEM((REM,), jnp.int32),
           pltpu.VMEM((REM,), jnp.float32),
           pltpu.VMEM((REM,), jnp.float32),
           pltpu.VMEM((REM,), jnp.float32),
           pltpu.VMEM((REM, D), jnp.float32)]
        + [pltpu.VMEM((2, C // 2), jnp.int32),
           pltpu.VMEM((2, C // 2), jnp.int32)]
        + [pltpu.VMEM((ZR, D), jnp.float32),
           pltpu.VMEM((DZ,), jnp.float32),
           pltpu.VMEM_SHARED((N, D), jnp.float32),
           pltpu.VMEM_SHARED((N,), jnp.float32)]
        + [pltpu.SemaphoreType.DMA] * 7
    ),
)(_edge_body)


def _combine_body(p_ref, d_ref, o_ref):
    dsum = jnp.sum(d_ref[...], axis=1)
    r = 1.0 / (dsum + 1e-10)
    o_ref[...] = (p_ref[0] + p_ref[1]) * r[:, None]


def _combine(part, dT):
    blk = 2000
    grid = N // blk
    return pl.pallas_call(
        _combine_body,
        grid=(grid,),
        in_specs=[
            pl.BlockSpec((NC, blk, D), lambda i: (0, i, 0)),
            pl.BlockSpec((blk, NC), lambda i: (i, 0)),
        ],
        out_specs=pl.BlockSpec((blk, D), lambda i: (i, 0)),
        out_shape=jax.ShapeDtypeStruct((N, D), jnp.float32),
    )(part, dT)


def kernel(h, row, col, W, a):
    row32 = row.astype(jnp.int32)
    col32 = col.astype(jnp.int32)
    wh, s = _prep(h, W, a)
    s1 = s[:, 0]
    s2 = s[:, 1]
    part, dpart = _edge_kernel(wh, s1, s2, row32, col32)
    return _combine(part, dpart.reshape(NC, N).T)


# speedup vs baseline: 1.0461x; 1.0003x over previous
import functools

import jax
import jax.numpy as jnp
from jax import lax
from jax.experimental import pallas as pl
from jax.experimental.pallas import tpu as pltpu
from jax.experimental.pallas import tpu_sc as plsc

N = 10000
E = 320000
D = 128
NC = 2
NS = 16
NW = NC * NS
EPW = E // NW
C = 128
NFULL = EPW // C
PAIRS = NFULL // 2
REM = EPW - NFULL * C
RPT = N // 10
ZR = 40
DZ = N // 5


def _prep_body(h_ref, w_ref, a_ref, wh_ref, s_ref):
    wh = jnp.dot(h_ref[...], w_ref[...], preferred_element_type=jnp.float32)
    wh_ref[...] = wh
    ar = jnp.concatenate([a_ref[:D, :], a_ref[D:, :]], axis=1)
    s_ref[...] = jnp.dot(wh, ar, preferred_element_type=jnp.float32)


def _prep(h, W, a):
    blk = 2000
    grid = N // blk
    return pl.pallas_call(
        _prep_body,
        grid=(grid,),
        in_specs=[
            pl.BlockSpec((blk, D), lambda i: (i, 0)),
            pl.BlockSpec((D, D), lambda i: (0, 0)),
            pl.BlockSpec((2 * D, 1), lambda i: (0, 0)),
        ],
        out_specs=[
            pl.BlockSpec((blk, D), lambda i: (i, 0)),
            pl.BlockSpec((blk, 2), lambda i: (i, 0)),
        ],
        out_shape=[
            jax.ShapeDtypeStruct((N, D), jnp.float32),
            jax.ShapeDtypeStruct((N, 2), jnp.float32),
        ],
    )(h, W, a)


def _edge_body(wh_hbm, s1_hbm, s2_hbm, row_hbm, col_hbm, part_hbm, dpart_hbm,
               idxr_a, idxc_a, s1g_a, s2g_a, ee_a, wrows_a,
               idxr_b, idxc_b, s1g_b, s2g_b, ee_b, wrows_b,
               idxr_r, idxc_r, s1g_r, s2g_r, ee_r, wrows_r,
               sidxr_a, sidxr_b, zrows, zd, acc, dacc,
               semg_a, semg_b, semi_a, semi_b, sems_a, sems_b, sems_r):
    cid = lax.axis_index("c")
    sid = lax.axis_index("s")
    wid = sid * NC + cid

    def _zero_accumulators():
        def _zrow(i, _):
            for j in range(D // 16):
                zrows[i, pl.ds(j * 16, 16)] = jnp.zeros((16,), jnp.float32)
            return 0
        lax.fori_loop(0, ZR, _zrow, 0)

        def _zd(i, _):
            zd[pl.ds(i * 16, 16)] = jnp.zeros((16,), jnp.float32)
            return 0
        lax.fori_loop(0, DZ // 16, _zd, 0)

        @pl.when(sid < 10)
        def _():
            for r in range(RPT // ZR):
                pltpu.sync_copy(zrows, acc.at[pl.ds(sid * RPT + r * ZR, ZR)])

        @pl.when(sid < 5)
        def _():
            pltpu.sync_copy(zd, dacc.at[pl.ds(sid * DZ, DZ)])

    base = wid * EPW

    def _scale_span(ee, wrows, g0, g1):
        def _scale(g, _):
            ev = ee[pl.ds(g * 16, 16)]
            for e in range(16):
                v = lax.gather(
                    ev, jnp.full((16, 1), e, jnp.int32),
                    lax.GatherDimensionNumbers(
                        offset_dims=(), collapsed_slice_dims=(0,),
                        start_index_map=(0,)),
                    (1,), mode=lax.GatherScatterMode.PROMISE_IN_BOUNDS)
                eidx = g * 16 + e
                for j in range(D // 16):
                    wrows[eidx, pl.ds(j * 16, 16)] = (
                        wrows[eidx, pl.ds(j * 16, 16)] * v)
            return 0
        lax.fori_loop(g0, g1, _scale, 0)

    def _compute_scatter(n, idxr, s1g, s2g, ee, wrows, sems):
        for j in range(n // 16):
            t = s1g[pl.ds(j * 16, 16)] + s2g[pl.ds(j * 16, 16)]
            ee[pl.ds(j * 16, 16)] = jnp.exp(jnp.maximum(t, 0.2 * t))
        pltpu.async_copy(ee, dacc.at[idxr], sems, add=True)
        _scale_span(ee, wrows, 0, n // 16)
        pltpu.async_copy(wrows, acc.at[idxr], sems, add=True)

    H = C // 2

    def _compute_scatter2(idxr2, s1g, s2g, ee, wrows, sems):
        for j in range(C // 16):
            t = s1g[pl.ds(j * 16, 16)] + s2g[pl.ds(j * 16, 16)]
            ee[pl.ds(j * 16, 16)] = jnp.exp(jnp.maximum(t, 0.2 * t))
        pltpu.async_copy(ee.at[pl.ds(0, H)], dacc.at[idxr2.at[0]], sems,
                         add=True)
        pltpu.async_copy(ee.at[pl.ds(H, H)], dacc.at[idxr2.at[1]], sems,
                         add=True)
        _scale_span(ee, wrows, 0, H // 16)
        pltpu.async_copy(wrows.at[pl.ds(0, H)], acc.at[idxr2.at[0]], sems,
                         add=True)
        _scale_span(ee, wrows, H // 16, C // 16)
        pltpu.async_copy(wrows.at[pl.ds(H, H)], acc.at[idxr2.at[1]], sems,
                         add=True)

    def _fire_idx(k, idxr, idxc, semi):
        off = base + k * C
        pltpu.async_copy(row_hbm.at[pl.ds(off, C)], idxr, semi)
        pltpu.async_copy(col_hbm.at[pl.ds(off, C)], idxc, semi)

    def _wait_idx(idxr, idxc, semi):
        pltpu.make_async_copy(row_hbm.at[pl.ds(0, C)], idxr, semi).wait()
        pltpu.make_async_copy(col_hbm.at[pl.ds(0, C)], idxc, semi).wait()

    def _fire_gathers(idxr, idxc, s1g, s2g, wrows, semg):
        pltpu.async_copy(s1_hbm.at[idxr], s1g, semg)
        pltpu.async_copy(s2_hbm.at[idxc], s2g, semg)
        pltpu.async_copy(wh_hbm.at[idxc.at[pl.ds(0, H)]],
                         wrows.at[pl.ds(0, H)], semg)
        pltpu.async_copy(wh_hbm.at[idxc.at[pl.ds(H, H)]],
                         wrows.at[pl.ds(H, H)], semg)

    def _wait_gathers(s1g, s2g, wrows, semg):
        pltpu.make_async_copy(s1_hbm.at[pl.ds(0, C)], s1g, semg).wait()
        pltpu.make_async_copy(s2_hbm.at[pl.ds(0, C)], s2g, semg).wait()
        pltpu.make_async_copy(wh_hbm.at[pl.ds(0, C)], wrows, semg).wait()

    def _wait_scatters(ee, wrows, sems, n):
        pltpu.make_async_copy(s1_hbm.at[pl.ds(0, n)], ee, sems).wait()
        pltpu.make_async_copy(wh_hbm.at[pl.ds(0, n)], wrows, sems).wait()

    offr = base + NFULL * C
    pltpu.sync_copy(row_hbm.at[pl.ds(offr, REM)], idxr_r)
    pltpu.sync_copy(col_hbm.at[pl.ds(offr, REM)], idxc_r)
    pltpu.sync_copy(s1_hbm.at[idxr_r], s1g_r)
    pltpu.sync_copy(s2_hbm.at[idxc_r], s2g_r)
    pltpu.sync_copy(wh_hbm.at[idxc_r], wrows_r)
    _fire_idx(0, idxr_a, idxc_a, semi_a)
    _wait_idx(idxr_a, idxc_a, semi_a)
    _fire_gathers(idxr_a, idxc_a, s1g_a, s2g_a, wrows_a, semg_a)
    _fire_idx(1, idxr_b, idxc_b, semi_b)

    _zero_accumulators()
    plsc.subcore_barrier()
    _compute_scatter(REM, idxr_r, s1g_r, s2g_r, ee_r, wrows_r, sems_r)

    def _copy_idx(idxr, sidxr):
        for j in range(H // 16):
            sidxr[0, pl.ds(j * 16, 16)] = idxr[pl.ds(j * 16, 16)]
            sidxr[1, pl.ds(j * 16, 16)] = idxr[pl.ds(H + j * 16, 16)]

    def _pair(i, _):
        ka = 2 * i
        kb = ka + 1
        _wait_gathers(s1g_a, s2g_a, wrows_a, semg_a)
        _copy_idx(idxr_a, sidxr_a)
        _wait_idx(idxr_b, idxc_b, semi_b)

        @pl.when(i > 0)
        def _():
            _wait_scatters(ee_b, wrows_b, sems_b, C)
        _fire_gathers(idxr_b, idxc_b, s1g_b, s2g_b, wrows_b, semg_b)

        @pl.when(i < PAIRS - 1)
        def _():
            _fire_idx(ka + 2, idxr_a, idxc_a, semi_a)
        _compute_scatter2(sidxr_a, s1g_a, s2g_a, ee_a, wrows_a, sems_a)
        _wait_gathers(s1g_b, s2g_b, wrows_b, semg_b)
        _copy_idx(idxr_b, sidxr_b)
        _wait_scatters(ee_a, wrows_a, sems_a, C)

        @pl.when(i < PAIRS - 1)
        def _():
            _wait_idx(idxr_a, idxc_a, semi_a)
            _fire_gathers(idxr_a, idxc_a, s1g_a, s2g_a, wrows_a, semg_a)
            _fire_idx(kb + 2, idxr_b, idxc_b, semi_b)
        _compute_scatter2(sidxr_b, s1g_b, s2g_b, ee_b, wrows_b, sems_b)
        return 0

    lax.fori_loop(0, PAIRS, _pair, 0)
    _wait_scatters(ee_b, wrows_b, sems_b, C)
    _wait_scatters(ee_r, wrows_r, sems_r, REM)
    plsc.subcore_barrier()

    @pl.when(sid < 10)
    def _():
        for r in range(RPT // ZR):
            off = sid * RPT + r * ZR
            pltpu.sync_copy(acc.at[pl.ds(off, ZR)], zrows)
            pltpu.sync_copy(zrows, part_hbm.at[cid, pl.ds(off, ZR)])

    @pl.when(sid < 5)
    def _():
        pltpu.sync_copy(dacc.at[pl.ds(sid * DZ, DZ)], zd)
        pltpu.sync_copy(zd, dpart_hbm.at[pl.ds(cid * N + sid * DZ, DZ)])


_edge_kernel = functools.partial(
    pl.kernel,
    out_type=[
        jax.ShapeDtypeStruct((NC, N, D), jnp.float32),
        jax.ShapeDtypeStruct((NC * N,), jnp.float32),
    ],
    mesh=plsc.VectorSubcoreMesh(core_axis_name="c", subcore_axis_name="s"),
    scratch_types=(
        [pltpu.VMEM((C,), jnp.int32),
         pltpu.VMEM((C,), jnp.int32),
         pltpu.VMEM((C,), jnp.float32),
         pltpu.VMEM((C,), jnp.float32),
         pltpu.VMEM((C,), jnp.float32),
         pltpu.VMEM((C, D), jnp.float32)]
        + [pltpu.VMEM((C,), jnp.int32),
           pltpu.VMEM((C,), jnp.int32),
           pltpu.VMEM((C,), jnp.float32),
           pltpu.VMEM((C,), jnp.float32),
           pltpu.VMEM((C,), jnp.float32),
           pltpu.VMEM((C, D), jnp.float32)]
        + [pltpu.VMEM((REM,), jnp.int32),
           pltpu.VMEM((REM,), jnp.int32),
           pltpu.VMEM((REM,), jnp.float32),
           pltpu.VMEM((REM,), jnp.float32),
           pltpu.VMEM((REM,), jnp.float32),
           pltpu.VMEM((REM, D), jnp.float32)]
        + [pltpu.VMEM((2, C // 2), jnp.int32),
           pltpu.VMEM((2, C // 2), jnp.int32)]
        + [pltpu.VMEM((ZR, D), jnp.float32),
           pltpu.VMEM((DZ,), jnp.float32),
           pltpu.VMEM_SHARED((N, D), jnp.float32),
           pltpu.VMEM_SHARED((N,), jnp.float32)]
        + [pltpu.SemaphoreType.DMA] * 7
    ),
)(_edge_body)


def _combine_body(p_ref, d_ref, o_ref):
    dsum = jnp.sum(d_ref[...], axis=1)
    r = 1.0 / (dsum + 1e-10)
    o_ref[...] = (p_ref[0] + p_ref[1]) * r[:, None]


def _combine(part, dT):
    blk = 2000
    grid = N // blk
    return pl.pallas_call(
        _combine_body,
        grid=(grid,),
        in_specs=[
            pl.BlockSpec((NC, blk, D), lambda i: (0, i, 0)),
            pl.BlockSpec((blk, NC), lambda i: (i, 0)),
        ],
        out_specs=pl.BlockSpec((blk, D), lambda i: (i, 0)),
        out_shape=jax.ShapeDtypeStruct((N, D), jnp.float32),
    )(part, dT)


def kernel(h, row, col, W, a):
    row32 = row.astype(jnp.int32)
    col32 = col.astype(jnp.int32)
    wh, s = _prep(h, W, a)
    s1 = s[:, 0]
    s2 = s[:, 1]
    part, dpart = _edge_kernel(wh, s1, s2, row32, col32)
    return _combine(part, dpart.reshape(NC, N).T)
